# Initial kernel scaffold; baseline (speedup 1.0000x reference)
import jax
import jax.numpy as jnp
from jax.experimental import pallas as pl


def kernel(x, edge_index, edge_type, weight, w_comp, self_loop_weight):
    raise NotImplementedError("write your pallas kernel here")


# speedup vs baseline: 8.6122x; 8.6122x over previous
import functools

import jax
import jax.numpy as jnp
from jax import lax
from jax.experimental import pallas as pl
from jax.experimental.pallas import tpu as pltpu
from jax.experimental.pallas import tpu_sc as plsc

N = 10000
E = 320000
DIN = 128
DOUT = 128
R = 8
NB = 4

NPAD = 10240
NPLANES = R + 1
NC = 2
NS = 16
NW = NC * NS
EPW = 10240
EPAD = NW * EPW
CHUNK = 128
NCHUNK = EPW // CHUNK
ROWS_PER_TILE = NPAD // NS


_BLK = 1024


def _transform_body(wc_ref, wext_ref, x_ref, out_ref):
    r = pl.program_id(0)
    w = wc_ref[r, 0] * wext_ref[0]
    for b in range(1, NB + 1):
        w += wc_ref[r, b] * wext_ref[b]
    out_ref[0] = jnp.dot(x_ref[...], w, preferred_element_type=jnp.float32)


def _transform(xpad, w_ext, wc_ext):
    return pl.pallas_call(
        _transform_body,
        grid=(NPLANES, NPAD // _BLK),
        in_specs=[
            pl.BlockSpec(memory_space=pltpu.SMEM),
            pl.BlockSpec((NB + 1, DIN, DOUT), lambda r, j: (0, 0, 0)),
            pl.BlockSpec((_BLK, DIN), lambda r, j: (j, 0)),
        ],
        out_specs=pl.BlockSpec((1, _BLK, DOUT), lambda r, j: (r, j, 0)),
        out_shape=jax.ShapeDtypeStruct((NPLANES, NPAD, DOUT), jnp.float32),
    )(wc_ext, w_ext, xpad)


def _sc_body(table_hbm, src_hbm, type_hbm, dst2_hbm, zeros_hbm,
             p0_hbm, p1_hbm,
             acc, src_v, type_v, dst2_v, gidx_v, rows_v, sem):
    c = lax.axis_index("c")
    s = lax.axis_index("s")
    wid = s * NC + c
    base = wid * EPW

    pltpu.sync_copy(src_hbm.at[pl.ds(base, EPW)], src_v)
    pltpu.sync_copy(type_hbm.at[pl.ds(base, EPW)], type_v)
    pltpu.sync_copy(dst2_hbm.at[pl.ds(wid * NCHUNK, NCHUNK)], dst2_v)

    row0 = s * ROWS_PER_TILE

    @pl.when(c == 0)
    def _():
        pltpu.sync_copy(table_hbm.at[pl.ds(R * NPAD + row0, ROWS_PER_TILE)],
                        acc.at[pl.ds(row0, ROWS_PER_TILE)])

    @pl.when(c != 0)
    def _():
        pltpu.sync_copy(zeros_hbm, acc.at[pl.ds(row0, ROWS_PER_TILE)])

    plsc.subcore_barrier()

    def chunk_body(ci, carry):
        off = ci * CHUNK
        for j in range(CHUNK // 16):
            t = type_v[pl.ds(off + j * 16, 16)]
            sv = src_v[pl.ds(off + j * 16, 16)]
            gidx_v[pl.ds(j * 16, 16)] = t * NPAD + sv
        pltpu.async_copy(table_hbm.at[gidx_v], rows_v, sem).wait()
        pltpu.sync_copy(rows_v, acc.at[dst2_v.at[ci]], add=True)
        return carry

    lax.fori_loop(0, NCHUNK, chunk_body, 0)
    plsc.subcore_barrier()

    @pl.when(c == 0)
    def _():
        pltpu.sync_copy(acc.at[pl.ds(row0, ROWS_PER_TILE)],
                        p0_hbm.at[pl.ds(row0, ROWS_PER_TILE)])

    @pl.when(c != 0)
    def _():
        pltpu.sync_copy(acc.at[pl.ds(row0, ROWS_PER_TILE)],
                        p1_hbm.at[pl.ds(row0, ROWS_PER_TILE)])


def _sc_aggregate(table, srcp, typep, dst2, zeros):
    mesh = plsc.VectorSubcoreMesh(core_axis_name="c", subcore_axis_name="s",
                                  num_cores=NC, num_subcores=NS)
    f = pl.kernel(
        _sc_body,
        out_type=[
            jax.ShapeDtypeStruct((NPAD, DOUT), jnp.float32),
            jax.ShapeDtypeStruct((NPAD, DOUT), jnp.float32),
        ],
        mesh=mesh,
        scratch_types=[
            pltpu.VMEM_SHARED((NPAD, DOUT), jnp.float32),
            pltpu.VMEM((EPW,), jnp.int32),
            pltpu.VMEM((EPW,), jnp.int32),
            pltpu.VMEM((NCHUNK, CHUNK), jnp.int32),
            pltpu.VMEM((CHUNK,), jnp.int32),
            pltpu.VMEM((CHUNK, DOUT), jnp.float32),
            pltpu.SemaphoreType.DMA,
        ],
    )
    return f(table, srcp, typep, dst2, zeros)


def _combine_body(a_ref, b_ref, o_ref):
    o_ref[...] = jnp.maximum(a_ref[...] + b_ref[...], 0.0)


def _combine(p0, p1):
    return pl.pallas_call(
        _combine_body,
        grid=(NPAD // _BLK,),
        in_specs=[
            pl.BlockSpec((_BLK, DOUT), lambda i: (i, 0)),
            pl.BlockSpec((_BLK, DOUT), lambda i: (i, 0)),
        ],
        out_specs=pl.BlockSpec((_BLK, DOUT), lambda i: (i, 0)),
        out_shape=jax.ShapeDtypeStruct((NPAD, DOUT), jnp.float32),
    )(p0, p1)


@jax.jit
def kernel(x, edge_index, edge_type, weight, w_comp, self_loop_weight):
    xpad = jnp.pad(x, ((0, NPAD - N), (0, 0)))
    w_ext = jnp.concatenate([weight, self_loop_weight[None]], axis=0)
    wc_ext = jnp.zeros((NPLANES, NB + 1), jnp.float32)
    wc_ext = wc_ext.at[:R, :NB].set(w_comp).at[R, NB].set(1.0)

    table3 = _transform(xpad, w_ext, wc_ext)
    table = table3.reshape(NPLANES * NPAD, DOUT)

    srcp = jnp.pad(edge_index[0], (0, EPAD - E))
    typep = jnp.pad(edge_type, (0, EPAD - E))
    dstp = jnp.pad(edge_index[1], (0, EPAD - E),
                   constant_values=NPAD - 1)
    dst2 = dstp.reshape(EPAD // CHUNK, CHUNK)
    zeros = jnp.zeros((ROWS_PER_TILE, DOUT), jnp.float32)

    p0, p1 = _sc_aggregate(table, srcp, typep, dst2, zeros)
    out = _combine(p0, p1)
    return out[:N]
